# Initial kernel scaffold; baseline (speedup 1.0000x reference)
import jax
import jax.numpy as jnp
from jax.experimental import pallas as pl


def kernel(x_0, x_1, edge_index_0, edge_index_1, target_idx_0, target_idx_1, W_0, attn_l_0, attn_r_0, b_0, W_1, attn_l_1, attn_r_1, b_1):
    raise NotImplementedError("write your pallas kernel here")


# speedup vs baseline: 52.8976x; 52.8976x over previous
import functools

import jax
import jax.numpy as jnp
from jax import lax
from jax.experimental import pallas as pl
from jax.experimental.pallas import tpu as pltpu
from jax.experimental.pallas import tpu_sc as plsc

N_NODES = 10000
N_EDGES = 320000
IN_DIM = 128
HID = 16
HEADS = 8
F = HEADS * HID
N_TGT = 5000

NC = 2
NS = 16

EPT = N_EDGES // NS
MC = 160
NMC = EPT // MC
MIC = 80
NMIC = MC // MIC

T_PAD = 5120
TPT = T_PAD // NS
NTR = TPT // MC

NZC = N_NODES // MC
NZT = N_NODES - NZC * MC

_BLK = 1000
_NB = N_NODES // _BLK


def _tc_body(x_ref, w_ref, al_ref, ar_ref, h_ref, el_ref, er_ref, m_ref):
    i = pl.program_id(1)
    h = jnp.dot(x_ref[0], w_ref[0], preferred_element_type=jnp.float32)
    el = jnp.dot(h, al_ref[0], preferred_element_type=jnp.float32)
    er = jnp.dot(h, ar_ref[0], preferred_element_type=jnp.float32)
    h_ref[0] = h
    el_ref[0] = jnp.concatenate([el, el], axis=1)
    er_ref[0] = jnp.concatenate([er, er], axis=1)
    cur = jnp.concatenate(
        [jnp.max(el, axis=0, keepdims=True), jnp.max(er, axis=0, keepdims=True)],
        axis=1)

    @pl.when(i == 0)
    def _():
        m_ref[0] = cur

    @pl.when(i > 0)
    def _():
        m_ref[0] = jnp.maximum(m_ref[0], cur)

    @pl.when(i == _NB - 1)
    def _():
        acc = m_ref[0]
        s = acc[:, 0:HEADS] + acc[:, HEADS:2 * HEADS]
        mf = jnp.where(s > 0.0, s, 0.2 * s)
        m_ref[0] = jnp.concatenate([mf, mf], axis=1)


def _tc_call(x_all, w_all, al, ar):
    return pl.pallas_call(
        _tc_body,
        grid=(2, _NB),
        in_specs=[
            pl.BlockSpec((1, _BLK, IN_DIM), lambda m, i: (m, i, 0)),
            pl.BlockSpec((1, IN_DIM, F), lambda m, i: (m, 0, 0)),
            pl.BlockSpec((1, F, HEADS), lambda m, i: (m, 0, 0)),
            pl.BlockSpec((1, F, HEADS), lambda m, i: (m, 0, 0)),
        ],
        out_specs=[
            pl.BlockSpec((1, _BLK, F), lambda m, i: (m, i, 0)),
            pl.BlockSpec((1, _BLK, 16), lambda m, i: (m, i, 0)),
            pl.BlockSpec((1, _BLK, 16), lambda m, i: (m, i, 0)),
            pl.BlockSpec((1, 1, 16), lambda m, i: (m, 0, 0)),
        ],
        out_shape=[
            jax.ShapeDtypeStruct((2, N_NODES, F), jnp.float32),
            jax.ShapeDtypeStruct((2, N_NODES, 16), jnp.float32),
            jax.ShapeDtypeStruct((2, N_NODES, 16), jnp.float32),
            jax.ShapeDtypeStruct((2, 1, 16), jnp.float32),
        ],
    )(x_all, w_all, al, ar)


def _sc_body(h2, ela, era, edges, tgts, mvec, bias2, out,
             acc_sp, den_sp,
             src_v, srch_v, dst_v, dsta_v, h_g, elg, erg, exb, mv, bias_v):
    cid = lax.axis_index("c")
    sid = lax.axis_index("s")

    pltpu.sync_copy(mvec.at[pl.ds(16 * cid, 16)], mv)
    pltpu.sync_copy(bias2.at[pl.ds(F * cid, F)], bias_v)

    @pl.loop(0, MC)
    def _(c):
        exb[c, :] = jnp.zeros((16,), jnp.float32)
        for j in range(F // 16):
            h_g[c, pl.ds(j * 16, 16)] = jnp.zeros((16,), jnp.float32)

    for r in range(4):
        ck = sid + NS * r

        @pl.when(ck < NZC)
        def _():
            rows = pl.ds(ck * MC, MC)
            pltpu.sync_copy(h_g, acc_sp.at[rows])
            pltpu.sync_copy(exb, den_sp.at[rows])

    @pl.when(sid == NS - 1)
    def _():
        rows = pl.ds(NZC * MC, NZT)
        pltpu.sync_copy(h_g.at[pl.ds(0, NZT)], acc_sp.at[rows])
        pltpu.sync_copy(exb.at[pl.ds(0, NZT)], den_sp.at[rows])

    plsc.subcore_barrier()

    off = cid * N_NODES
    ebase0 = sid * EPT

    @pl.loop(0, NMC)
    def _(k):
        eb = ebase0 + k * MC
        src_base = 2 * cid * N_EDGES + eb
        dst_base = (2 * cid + 1) * N_EDGES + eb
        for j in range(NMIC):
            pltpu.sync_copy(edges.at[pl.ds(src_base + j * MIC, MIC)],
                            src_v.at[j])
            pltpu.sync_copy(edges.at[pl.ds(dst_base + j * MIC, MIC)],
                            dst_v.at[j])
        for j in range(NMIC):
            for i in range(MIC // 16):
                sl = pl.ds(i * 16, 16)
                srch_v[j, sl] = src_v[j, sl] + off
                dsta_v[j, sl] = dst_v[j, sl] + off
        for j in range(NMIC):
            rows = pl.ds(j * MIC, MIC)
            pltpu.sync_copy(h2.at[srch_v.at[j]], h_g.at[rows])
            pltpu.sync_copy(ela.at[srch_v.at[j]], elg.at[rows])
            pltpu.sync_copy(era.at[dsta_v.at[j]], erg.at[rows])

        @pl.loop(0, MC)
        def _(c):
            e = elg[c, :] + erg[c, :]
            e = jnp.where(e > 0.0, e, 0.2 * e)
            ex = jnp.exp(e - mv[...])
            exb[c, :] = ex
            for hh in range(HEADS):
                sl = pl.ds(hh * HID, HID)
                h_g[c, sl] = h_g[c, sl] * ex[hh]

        for j in range(NMIC):
            rows = pl.ds(j * MIC, MIC)
            pltpu.sync_copy(exb.at[rows], den_sp.at[dst_v.at[j]], add=True)
            pltpu.sync_copy(h_g.at[rows], acc_sp.at[dst_v.at[j]], add=True)

    plsc.subcore_barrier()

    tb = sid * TPT
    tgt_base = cid * T_PAD + tb
    for r in range(NTR):
        rb = r * MC
        for j in range(NMIC):
            pltpu.sync_copy(
                tgts.at[pl.ds(tgt_base + rb + j * MIC, MIC)], src_v.at[j])
        for j in range(NMIC):
            rows = pl.ds(j * MIC, MIC)
            pltpu.sync_copy(acc_sp.at[src_v.at[j]], h_g.at[rows])
            pltpu.sync_copy(den_sp.at[src_v.at[j]], exb.at[rows])

        @pl.loop(0, MC)
        def _(t):
            dv = jnp.maximum(exb[t, :], 1e-9)
            for hh in range(HEADS):
                sl = pl.ds(hh * HID, HID)
                v = h_g[t, sl] / dv[hh] + bias_v[sl]
                v = jnp.where(v > 0.0, v, jnp.exp(v) - 1.0)
                h_g[t, sl] = v

        pltpu.sync_copy(h_g, out.at[cid, pl.ds(tb + rb, MC)])


def _sc_call(h2, ela, era, edges, tgts, mvec, bias2):
    mesh = plsc.VectorSubcoreMesh(core_axis_name="c", subcore_axis_name="s")
    kfn = pl.kernel(
        _sc_body,
        out_type=jax.ShapeDtypeStruct((2, T_PAD, F), jnp.float32),
        mesh=mesh,
        compiler_params=pltpu.CompilerParams(use_tc_tiling_on_sc=False),
        scratch_types=[
            pltpu.VMEM_SHARED((N_NODES, F), jnp.float32),
            pltpu.VMEM_SHARED((N_NODES, 16), jnp.float32),
            pltpu.VMEM((NMIC, MIC), jnp.int32),
            pltpu.VMEM((NMIC, MIC), jnp.int32),
            pltpu.VMEM((NMIC, MIC), jnp.int32),
            pltpu.VMEM((NMIC, MIC), jnp.int32),
            pltpu.VMEM((MC, F), jnp.float32),
            pltpu.VMEM((MC, 16), jnp.float32),
            pltpu.VMEM((MC, 16), jnp.float32),
            pltpu.VMEM((MC, 16), jnp.float32),
            pltpu.VMEM((16,), jnp.float32),
            pltpu.VMEM((F,), jnp.float32),
        ],
    )
    return kfn(h2, ela, era, edges, tgts, mvec, bias2)


def _attn_mat(a):
    eye = jnp.eye(HEADS, dtype=jnp.float32)
    return (a[:, :, None] * eye[:, None, :]).reshape(F, HEADS)


def kernel(x_0, x_1, edge_index_0, edge_index_1, target_idx_0, target_idx_1,
           W_0, attn_l_0, attn_r_0, b_0, W_1, attn_l_1, attn_r_1, b_1):
    x_all = jnp.stack([x_0, x_1])
    w_all = jnp.stack([W_0, W_1])
    al = jnp.stack([_attn_mat(attn_l_0), _attn_mat(attn_l_1)])
    ar = jnp.stack([_attn_mat(attn_r_0), _attn_mat(attn_r_1)])

    h3, el3, er3, m3 = _tc_call(x_all, w_all, al, ar)
    h2 = h3.reshape(2 * N_NODES, F)
    ela = el3.reshape(2 * N_NODES, 16)
    era = er3.reshape(2 * N_NODES, 16)
    mvec = m3.reshape(32)

    edges = jnp.concatenate([edge_index_0, edge_index_1], axis=0)
    edges = edges.astype(jnp.int32).reshape(4 * N_EDGES)
    pad = jnp.zeros((T_PAD - N_TGT,), jnp.int32)
    tgts = jnp.concatenate([
        target_idx_0.astype(jnp.int32), pad,
        target_idx_1.astype(jnp.int32), pad,
    ])
    bias2 = jnp.concatenate([b_0, b_1])

    out = _sc_call(h2, ela, era, edges, tgts, mvec, bias2)
    return (out[0, :N_TGT], out[1, :N_TGT])


# speedup vs baseline: 86.3740x; 1.6329x over previous
import functools

import jax
import jax.numpy as jnp
from jax import lax
from jax.experimental import pallas as pl
from jax.experimental.pallas import tpu as pltpu
from jax.experimental.pallas import tpu_sc as plsc

N_NODES = 10000
N_EDGES = 320000
IN_DIM = 128
HID = 16
HEADS = 8
F = HEADS * HID
N_TGT = 5000

NC = 2
NS = 16

EPT = N_EDGES // NS
MC = 80
NMC = EPT // MC
IW = 3 * MC

T_PAD = 5120
TPT = T_PAD // NS
NTR = TPT // MC

NZC = N_NODES // MC

_BLK = 1000
_NB = N_NODES // _BLK


def _tc_body(x_ref, w_ref, al_ref, ar_ref, h_ref, el_ref, er_ref, m_ref):
    i = pl.program_id(1)
    h = jnp.dot(x_ref[0], w_ref[0], preferred_element_type=jnp.float32)
    el = jnp.dot(h, al_ref[0], preferred_element_type=jnp.float32)
    er = jnp.dot(h, ar_ref[0], preferred_element_type=jnp.float32)
    h_ref[0] = h
    el_ref[0] = jnp.concatenate([el, el], axis=1)
    er_ref[0] = jnp.concatenate([er, er], axis=1)
    cur = jnp.concatenate(
        [jnp.max(el, axis=0, keepdims=True), jnp.max(er, axis=0, keepdims=True)],
        axis=1)

    @pl.when(i == 0)
    def _():
        m_ref[0] = cur

    @pl.when(i > 0)
    def _():
        m_ref[0] = jnp.maximum(m_ref[0], cur)

    @pl.when(i == _NB - 1)
    def _():
        acc = m_ref[0]
        s = acc[:, 0:HEADS] + acc[:, HEADS:2 * HEADS]
        mf = jnp.where(s > 0.0, s, 0.2 * s)
        m_ref[0] = jnp.concatenate([mf, mf], axis=1)


def _tc_call(x_all, w_all, al, ar):
    return pl.pallas_call(
        _tc_body,
        grid=(2, _NB),
        in_specs=[
            pl.BlockSpec((1, _BLK, IN_DIM), lambda m, i: (m, i, 0)),
            pl.BlockSpec((1, IN_DIM, F), lambda m, i: (m, 0, 0)),
            pl.BlockSpec((1, F, HEADS), lambda m, i: (m, 0, 0)),
            pl.BlockSpec((1, F, HEADS), lambda m, i: (m, 0, 0)),
        ],
        out_specs=[
            pl.BlockSpec((1, _BLK, F), lambda m, i: (m, i, 0)),
            pl.BlockSpec((1, _BLK, 16), lambda m, i: (m, i, 0)),
            pl.BlockSpec((1, _BLK, 16), lambda m, i: (m, i, 0)),
            pl.BlockSpec((1, 1, 16), lambda m, i: (m, 0, 0)),
        ],
        out_shape=[
            jax.ShapeDtypeStruct((2, N_NODES, F), jnp.float32),
            jax.ShapeDtypeStruct((2, N_NODES, 16), jnp.float32),
            jax.ShapeDtypeStruct((2, N_NODES, 16), jnp.float32),
            jax.ShapeDtypeStruct((2, 1, 16), jnp.float32),
        ],
    )(x_all, w_all, al, ar)


def _sc_body(h2, ela, era, eidx, tgts, mvec, bias2, out,
             acc_sp, den_sp,
             ix0, ix1, hg0, hg1, el0, el1, er0, er1, exb, mv, bias_v,
             gs0, gs1):
    cid = lax.axis_index("c")
    sid = lax.axis_index("s")

    pltpu.sync_copy(mvec.at[pl.ds(16 * cid, 16)], mv)
    pltpu.sync_copy(bias2.at[pl.ds(F * cid, F)], bias_v)

    @pl.loop(0, MC)
    def _(c):
        exb[c, :] = jnp.zeros((16,), jnp.float32)
        for j in range(F // 16):
            hg0[c, pl.ds(j * 16, 16)] = jnp.zeros((16,), jnp.float32)

    for r in range(8):
        ck = sid * 8 + r

        @pl.when(ck < NZC)
        def _():
            rows = pl.ds(ck * MC, MC)
            pltpu.sync_copy(hg0, acc_sp.at[rows])
            pltpu.sync_copy(exb, den_sp.at[rows])

    plsc.subcore_barrier()

    ibase0 = (cid * NS + sid) * (NMC * IW)

    def load_idx(c, ix, gs):
        base = ibase0 + c * IW
        for r in range(3):
            pltpu.async_copy(eidx.at[pl.ds(base + r * MC, MC)], ix.at[r], gs)
        for r in range(3):
            pltpu.make_async_copy(eidx.at[pl.ds(0, MC)], ix.at[r], gs).wait()

    def fire_gathers(ix, hg, el, er, gs):
        pltpu.async_copy(ela.at[ix.at[0]], el, gs)
        pltpu.async_copy(era.at[ix.at[1]], er, gs)
        pltpu.async_copy(h2.at[ix.at[0]], hg, gs)

    def wait_gathers(hg, el, er, gs):
        pltpu.make_async_copy(ela.at[pl.ds(0, MC)], el, gs).wait()
        pltpu.make_async_copy(era.at[pl.ds(0, MC)], er, gs).wait()
        pltpu.make_async_copy(h2.at[pl.ds(0, MC)], hg, gs).wait()

    def compute_scatter(ix, hg, el, er):
        @pl.loop(0, MC)
        def _(c):
            e = el[c, :] + er[c, :]
            e = jnp.where(e > 0.0, e, 0.2 * e)
            ex = jnp.exp(e - mv[...])
            exb[c, :] = ex
            for hh in range(HEADS):
                sl = pl.ds(hh * HID, HID)
                hg[c, sl] = hg[c, sl] * ex[hh]

        pltpu.sync_copy(exb, den_sp.at[ix.at[2]], add=True)
        pltpu.sync_copy(hg, acc_sp.at[ix.at[2]], add=True)

    bufs = ((ix0, hg0, el0, er0, gs0), (ix1, hg1, el1, er1, gs1))

    for j in range(2):
        ix, hg, el, er, gs = bufs[j]
        load_idx(j, ix, gs)
        fire_gathers(ix, hg, el, er, gs)

    @pl.loop(0, NMC - 2, step=2)
    def _(g):
        for j in range(2):
            ix, hg, el, er, gs = bufs[j]
            wait_gathers(hg, el, er, gs)
            compute_scatter(ix, hg, el, er)
            load_idx(g + j + 2, ix, gs)
            fire_gathers(ix, hg, el, er, gs)

    for j in range(2):
        ix, hg, el, er, gs = bufs[j]
        wait_gathers(hg, el, er, gs)
        compute_scatter(ix, hg, el, er)

    plsc.subcore_barrier()

    tb = sid * TPT
    tgt_base = cid * T_PAD + tb
    for r in range(NTR):
        rb = r * MC
        pltpu.sync_copy(tgts.at[pl.ds(tgt_base + rb, MC)], ix0.at[0])
        pltpu.sync_copy(acc_sp.at[ix0.at[0]], hg0)
        pltpu.sync_copy(den_sp.at[ix0.at[0]], exb)

        @pl.loop(0, MC)
        def _(t):
            dv = jnp.maximum(exb[t, :], 1e-9)
            for hh in range(HEADS):
                sl = pl.ds(hh * HID, HID)
                v = hg0[t, sl] / dv[hh] + bias_v[sl]
                v = jnp.where(v > 0.0, v, jnp.exp(v) - 1.0)
                hg0[t, sl] = v

        pltpu.sync_copy(hg0, out.at[cid, pl.ds(tb + rb, MC)])


def _sc_call(h2, ela, era, eidx, tgts, mvec, bias2):
    mesh = plsc.VectorSubcoreMesh(core_axis_name="c", subcore_axis_name="s")
    kfn = pl.kernel(
        _sc_body,
        out_type=jax.ShapeDtypeStruct((2, T_PAD, F), jnp.float32),
        mesh=mesh,
        compiler_params=pltpu.CompilerParams(use_tc_tiling_on_sc=False),
        scratch_types=[
            pltpu.VMEM_SHARED((N_NODES, F), jnp.float32),
            pltpu.VMEM_SHARED((N_NODES, 16), jnp.float32),
            pltpu.VMEM((3, MC), jnp.int32),
            pltpu.VMEM((3, MC), jnp.int32),
            pltpu.VMEM((MC, F), jnp.float32),
            pltpu.VMEM((MC, F), jnp.float32),
            pltpu.VMEM((MC, 16), jnp.float32),
            pltpu.VMEM((MC, 16), jnp.float32),
            pltpu.VMEM((MC, 16), jnp.float32),
            pltpu.VMEM((MC, 16), jnp.float32),
            pltpu.VMEM((MC, 16), jnp.float32),
            pltpu.VMEM((16,), jnp.float32),
            pltpu.VMEM((F,), jnp.float32),
            pltpu.SemaphoreType.DMA,
            pltpu.SemaphoreType.DMA,
        ],
    )
    return kfn(h2, ela, era, eidx, tgts, mvec, bias2)


def _attn_mat(a):
    eye = jnp.eye(HEADS, dtype=jnp.float32)
    return (a[:, :, None] * eye[:, None, :]).reshape(F, HEADS)


def _idx_stream(edge_index, m):
    s = edge_index[0].astype(jnp.int32) + m * N_NODES
    d = edge_index[1].astype(jnp.int32)
    arr = jnp.stack(
        [s.reshape(NS, NMC, MC),
         (d + m * N_NODES).reshape(NS, NMC, MC),
         d.reshape(NS, NMC, MC)],
        axis=2)
    return arr.reshape(-1)


def kernel(x_0, x_1, edge_index_0, edge_index_1, target_idx_0, target_idx_1,
           W_0, attn_l_0, attn_r_0, b_0, W_1, attn_l_1, attn_r_1, b_1):
    x_all = jnp.stack([x_0, x_1])
    w_all = jnp.stack([W_0, W_1])
    al = jnp.stack([_attn_mat(attn_l_0), _attn_mat(attn_l_1)])
    ar = jnp.stack([_attn_mat(attn_r_0), _attn_mat(attn_r_1)])

    h3, el3, er3, m3 = _tc_call(x_all, w_all, al, ar)
    h2 = h3.reshape(2 * N_NODES, F)
    ela = el3.reshape(2 * N_NODES, 16)
    era = er3.reshape(2 * N_NODES, 16)
    mvec = m3.reshape(32)

    eidx = jnp.concatenate([_idx_stream(edge_index_0, 0),
                            _idx_stream(edge_index_1, 1)])
    pad = jnp.zeros((T_PAD - N_TGT,), jnp.int32)
    tgts = jnp.concatenate([
        target_idx_0.astype(jnp.int32), pad,
        target_idx_1.astype(jnp.int32), pad,
    ])
    bias2 = jnp.concatenate([b_0, b_1])

    out = _sc_call(h2, ela, era, eidx, tgts, mvec, bias2)
    return (out[0, :N_TGT], out[1, :N_TGT])
